# baseline (speedup 1.0000x reference)
import jax
import jax.numpy as jnp
from jax.experimental import pallas as pl
from jax.experimental.pallas import tpu as pltpu

_FB = 800
_W = _FB // 32
_WROWS = 32
_NBUF = 4
_NCH = 2
_CR = _FB // _NCH


def _p1(x_hbm, f_ref, bm_hbm, s_ref, x_buf, m_buf, in_sems, out_sems):
    i = pl.program_id(0)
    n = pl.num_programs(0)

    def start_in(step, slot):
        for c in range(_NCH):
            pltpu.make_async_copy(
                x_hbm.at[pl.ds(step * _FB + c * _CR, _CR), :],
                x_buf.at[slot, pl.ds(c * _CR, _CR), :],
                in_sems.at[slot]).start()

    def wait_in(step, slot):
        for c in range(_NCH):
            pltpu.make_async_copy(
                x_hbm.at[pl.ds(step * _FB + c * _CR, _CR), :],
                x_buf.at[slot, pl.ds(c * _CR, _CR), :],
                in_sems.at[slot]).wait()

    def out_copy(step, slot):
        return pltpu.make_async_copy(
            m_buf.at[slot], bm_hbm.at[pl.ds(step * _WROWS, _WROWS), :],
            out_sems.at[slot])

    @pl.when(i == 0)
    def _():
        start_in(0, 0)
        start_in(1, 1)
        start_in(2, 2)

    @pl.when(i + 3 < n)
    def _():
        start_in(i + 3, jax.lax.rem(i + 3, _NBUF))

    oslot = jax.lax.rem(i, 2)

    @pl.when(i >= 2)
    def _():
        out_copy(i - 2, oslot).wait()

    slot = jax.lax.rem(i, _NBUF)
    wait_in(i, slot)

    xv = x_buf[slot]
    nz = xv != 0.0
    fv = f_ref[...]
    b = xv.shape[1]
    masked = jnp.where(nz, fv, 0.0)
    part = jnp.sum(masked.reshape(_FB // 8, 8, b), axis=0)

    @pl.when(i == 0)
    def _():
        s_ref[...] = part

    @pl.when(i > 0)
    def _():
        s_ref[...] += part

    kvec = jax.lax.broadcasted_iota(jnp.int32, (_W, 32, b), 1)
    bits = nz.astype(jnp.int32).reshape(_W, 32, b)
    m_buf[oslot, 0:_W, :] = jnp.sum(bits << kvec, axis=1)

    out_copy(i, oslot).start()

    @pl.when(i == n - 1)
    def _():
        out_copy(i - 1, jax.lax.rem(i - 1, 2)).wait()
        out_copy(i, oslot).wait()


def _p2(bm_hbm, f_ref, s_ref, y_hbm, m_buf, y_buf, in_sems, out_sems):
    i = pl.program_id(0)
    n = pl.num_programs(0)

    def in_copy(step, slot):
        return pltpu.make_async_copy(
            bm_hbm.at[pl.ds(step * _WROWS, _WROWS), :], m_buf.at[slot],
            in_sems.at[slot])

    def start_out(step, slot):
        for c in range(_NCH):
            pltpu.make_async_copy(
                y_buf.at[slot, pl.ds(c * _CR, _CR), :],
                y_hbm.at[pl.ds(step * _FB + c * _CR, _CR), :],
                out_sems.at[slot]).start()

    def wait_out(step, slot):
        for c in range(_NCH):
            pltpu.make_async_copy(
                y_buf.at[slot, pl.ds(c * _CR, _CR), :],
                y_hbm.at[pl.ds(step * _FB + c * _CR, _CR), :],
                out_sems.at[slot]).wait()

    @pl.when(i == 0)
    def _():
        in_copy(0, 0).start()
        in_copy(1, 1).start()
        in_copy(2, 2).start()

    @pl.when(i + 3 < n)
    def _():
        in_copy(i + 3, jax.lax.rem(i + 3, _NBUF)).start()

    oslot = jax.lax.rem(i, _NBUF)

    @pl.when(i >= _NBUF)
    def _():
        wait_out(i - _NBUF, oslot)

    slot = jax.lax.rem(i, _NBUF)
    in_copy(i, slot).wait()

    sv = jnp.sum(s_ref[...], axis=0, keepdims=True)
    inv = jnp.where(sv == 0.0, 1.0, 1.0 / sv)
    fv = f_ref[...]
    scale = fv * inv
    wv = m_buf[slot][0:_W, :]
    b = wv.shape[1]
    kvec = jax.lax.broadcasted_iota(jnp.int32, (_W, 32, b), 1)
    expand = jnp.broadcast_to(wv.reshape(_W, 1, b), (_W, 32, b))
    mv = ((expand >> kvec) & 1).reshape(_FB, b) != 0
    y_buf[oslot] = jnp.where(mv, scale, 0.0)

    start_out(i, oslot)

    @pl.when(i == n - 1)
    def _():
        @pl.when(n >= 4)
        def _():
            wait_out(i - 3, jax.lax.rem(i - 3, _NBUF))

        @pl.when(n >= 3)
        def _():
            wait_out(i - 2, jax.lax.rem(i - 2, _NBUF))

        @pl.when(n >= 2)
        def _():
            wait_out(i - 1, jax.lax.rem(i - 1, _NBUF))

        wait_out(i, oslot)


def kernel(x, f):
    B, N = x.shape
    xt = x.T
    nsteps = N // _FB
    f2 = f.reshape(N, 1)
    cp = pltpu.CompilerParams(dimension_semantics=("arbitrary",))

    bitmap, s = pl.pallas_call(
        _p1,
        grid=(nsteps,),
        in_specs=[
            pl.BlockSpec(memory_space=pl.ANY),
            pl.BlockSpec((_FB, 1), lambda i: (i, 0)),
        ],
        out_specs=[
            pl.BlockSpec(memory_space=pl.ANY),
            pl.BlockSpec((8, B), lambda i: (0, 0)),
        ],
        out_shape=[
            jax.ShapeDtypeStruct((nsteps * _WROWS, B), jnp.int32),
            jax.ShapeDtypeStruct((8, B), jnp.float32),
        ],
        scratch_shapes=[
            pltpu.VMEM((_NBUF, _FB, B), jnp.float32),
            pltpu.VMEM((2, _WROWS, B), jnp.int32),
            pltpu.SemaphoreType.DMA((_NBUF,)),
            pltpu.SemaphoreType.DMA((2,)),
        ],
        compiler_params=cp,
    )(xt, f2)

    yt = pl.pallas_call(
        _p2,
        grid=(nsteps,),
        in_specs=[
            pl.BlockSpec(memory_space=pl.ANY),
            pl.BlockSpec((_FB, 1), lambda i: (i, 0)),
            pl.BlockSpec((8, B), lambda i: (0, 0)),
        ],
        out_specs=pl.BlockSpec(memory_space=pl.ANY),
        out_shape=jax.ShapeDtypeStruct((N, B), jnp.float32),
        scratch_shapes=[
            pltpu.VMEM((_NBUF, _WROWS, B), jnp.int32),
            pltpu.VMEM((_NBUF, _FB, B), jnp.float32),
            pltpu.SemaphoreType.DMA((_NBUF,)),
            pltpu.SemaphoreType.DMA((_NBUF,)),
        ],
        compiler_params=cp,
    )(bitmap, f2, s)

    return yt.T


# speedup vs baseline: 1.2408x; 1.2408x over previous
import jax
import jax.numpy as jnp
from jax.experimental import pallas as pl
from jax.experimental.pallas import tpu as pltpu

_FB = 800
_W = _FB // 32
_WROWS = 32
_NBUF = 4
_NCH = 2
_CR = _FB // _NCH


def _p1(x_hbm, f_ref, bm_hbm, s_ref, x_buf, m_buf, in_sems, out_sems):
    i = pl.program_id(0)
    n = pl.num_programs(0)

    def start_in(step, slot):
        for c in range(_NCH):
            pltpu.make_async_copy(
                x_hbm.at[pl.ds(step * _FB + c * _CR, _CR), :],
                x_buf.at[slot, pl.ds(c * _CR, _CR), :],
                in_sems.at[slot]).start()

    def wait_in(step, slot):
        for c in range(_NCH):
            pltpu.make_async_copy(
                x_hbm.at[pl.ds(step * _FB + c * _CR, _CR), :],
                x_buf.at[slot, pl.ds(c * _CR, _CR), :],
                in_sems.at[slot]).wait()

    def out_copy(step, slot):
        return pltpu.make_async_copy(
            m_buf.at[slot], bm_hbm.at[pl.ds(step * _WROWS, _WROWS), :],
            out_sems.at[slot])

    @pl.when(i == 0)
    def _():
        start_in(0, 0)
        start_in(1, 1)
        start_in(2, 2)

    @pl.when(i + 3 < n)
    def _():
        start_in(i + 3, jax.lax.rem(i + 3, _NBUF))

    oslot = jax.lax.rem(i, 2)

    @pl.when(i >= 2)
    def _():
        out_copy(i - 2, oslot).wait()

    slot = jax.lax.rem(i, _NBUF)
    wait_in(i, slot)

    xv = x_buf[slot]
    nz = xv != 0.0
    fv = jnp.transpose(f_ref[0], (1, 0))
    b = xv.shape[1]
    masked = jnp.where(nz, fv, 0.0)
    part = jnp.sum(masked.reshape(_FB // 8, 8, b), axis=0)

    @pl.when(i == 0)
    def _():
        s_ref[...] = part

    @pl.when(i > 0)
    def _():
        s_ref[...] += part

    kvec = jax.lax.broadcasted_iota(jnp.int32, (_W, 32, b), 1)
    bits = nz.astype(jnp.int32).reshape(_W, 32, b)
    m_buf[oslot, 0:_W, :] = jnp.sum(bits << kvec, axis=1)

    out_copy(i, oslot).start()

    @pl.when(i == n - 1)
    def _():
        out_copy(i - 1, jax.lax.rem(i - 1, 2)).wait()
        out_copy(i, oslot).wait()


def _p2(bm_hbm, f_ref, s_ref, y_hbm, m_buf, y_buf, in_sems, out_sems):
    i = pl.program_id(0)
    n = pl.num_programs(0)

    def in_copy(step, slot):
        return pltpu.make_async_copy(
            bm_hbm.at[pl.ds(step * _WROWS, _WROWS), :], m_buf.at[slot],
            in_sems.at[slot])

    def start_out(step, slot):
        for c in range(_NCH):
            pltpu.make_async_copy(
                y_buf.at[slot, pl.ds(c * _CR, _CR), :],
                y_hbm.at[pl.ds(step * _FB + c * _CR, _CR), :],
                out_sems.at[slot]).start()

    def wait_out(step, slot):
        for c in range(_NCH):
            pltpu.make_async_copy(
                y_buf.at[slot, pl.ds(c * _CR, _CR), :],
                y_hbm.at[pl.ds(step * _FB + c * _CR, _CR), :],
                out_sems.at[slot]).wait()

    @pl.when(i == 0)
    def _():
        in_copy(0, 0).start()
        in_copy(1, 1).start()
        in_copy(2, 2).start()

    @pl.when(i + 3 < n)
    def _():
        in_copy(i + 3, jax.lax.rem(i + 3, _NBUF)).start()

    oslot = jax.lax.rem(i, _NBUF)

    @pl.when(i >= _NBUF)
    def _():
        wait_out(i - _NBUF, oslot)

    slot = jax.lax.rem(i, _NBUF)
    in_copy(i, slot).wait()

    sv = jnp.sum(s_ref[...], axis=0, keepdims=True)
    inv = jnp.where(sv == 0.0, 1.0, 1.0 / sv)
    fv = jnp.transpose(f_ref[0], (1, 0))
    scale = fv * inv
    wv = m_buf[slot][0:_W, :]
    b = wv.shape[1]
    kvec = jax.lax.broadcasted_iota(jnp.int32, (_W, 32, b), 1)
    expand = jnp.broadcast_to(wv.reshape(_W, 1, b), (_W, 32, b))
    mv = ((expand >> kvec) & 1).reshape(_FB, b) != 0
    y_buf[oslot] = jnp.where(mv, scale, 0.0)

    start_out(i, oslot)

    @pl.when(i == n - 1)
    def _():
        @pl.when(n >= 4)
        def _():
            wait_out(i - 3, jax.lax.rem(i - 3, _NBUF))

        @pl.when(n >= 3)
        def _():
            wait_out(i - 2, jax.lax.rem(i - 2, _NBUF))

        @pl.when(n >= 2)
        def _():
            wait_out(i - 1, jax.lax.rem(i - 1, _NBUF))

        wait_out(i, oslot)


def kernel(x, f):
    B, N = x.shape
    xt = x.T
    nsteps = N // _FB
    f2 = f.reshape(nsteps, 1, _FB)
    cp = pltpu.CompilerParams(dimension_semantics=("arbitrary",))

    bitmap, s = pl.pallas_call(
        _p1,
        grid=(nsteps,),
        in_specs=[
            pl.BlockSpec(memory_space=pl.ANY),
            pl.BlockSpec((1, 1, _FB), lambda i: (i, 0, 0)),
        ],
        out_specs=[
            pl.BlockSpec(memory_space=pl.ANY),
            pl.BlockSpec((8, B), lambda i: (0, 0)),
        ],
        out_shape=[
            jax.ShapeDtypeStruct((nsteps * _WROWS, B), jnp.int32),
            jax.ShapeDtypeStruct((8, B), jnp.float32),
        ],
        scratch_shapes=[
            pltpu.VMEM((_NBUF, _FB, B), jnp.float32),
            pltpu.VMEM((2, _WROWS, B), jnp.int32),
            pltpu.SemaphoreType.DMA((_NBUF,)),
            pltpu.SemaphoreType.DMA((2,)),
        ],
        compiler_params=cp,
    )(xt, f2)

    yt = pl.pallas_call(
        _p2,
        grid=(nsteps,),
        in_specs=[
            pl.BlockSpec(memory_space=pl.ANY),
            pl.BlockSpec((1, 1, _FB), lambda i: (i, 0, 0)),
            pl.BlockSpec((8, B), lambda i: (0, 0)),
        ],
        out_specs=pl.BlockSpec(memory_space=pl.ANY),
        out_shape=jax.ShapeDtypeStruct((N, B), jnp.float32),
        scratch_shapes=[
            pltpu.VMEM((_NBUF, _WROWS, B), jnp.int32),
            pltpu.VMEM((_NBUF, _FB, B), jnp.float32),
            pltpu.SemaphoreType.DMA((_NBUF,)),
            pltpu.SemaphoreType.DMA((_NBUF,)),
        ],
        compiler_params=cp,
    )(bitmap, f2, s)

    return yt.T
